# Initial kernel scaffold; baseline (speedup 1.0000x reference)
import jax
import jax.numpy as jnp
from jax.experimental import pallas as pl


def kernel(x, edge_index, Wl0, Wr0, b0, Wl1, Wr1, b1, Wl2, Wr2, b2, Wl3, Wr3, b3):
    raise NotImplementedError("write your pallas kernel here")


# speedup vs baseline: 4.1079x; 4.1079x over previous
import functools

import jax
import jax.numpy as jnp
from jax import lax
from jax.experimental import pallas as pl
from jax.experimental.pallas import tpu as pltpu
from jax.experimental.pallas import tpu_sc as plsc

N = 10000
E = 160000
D = 256
DH = 128
NC = 2
NS = 16
NPAD = 10240

EPT = E // NS
G = 128
NFULL = EPT // G
TAIL = EPT - NFULL * G

EPT_C = E // (NC * NS)
NFULL_C = EPT_C // G
TAIL_C = EPT_C - NFULL_C * G

ROWS_T = NPAD // NS
ZR = 128

_mesh = plsc.VectorSubcoreMesh(core_axis_name="c", subcore_axis_name="s")


def _zero_fill(ref, nwords):
    z = jnp.zeros((16,), jnp.float32)

    def body(i, _):
        ref[pl.ds(i * 16, 16)] = z
        return 0

    lax.fori_loop(0, nwords // 16, body, 0, unroll=4)


def _zero_fill2(ref, rows, cols):
    z = jnp.zeros((16,), jnp.float32)
    cpr = cols // 16

    def body(i, _):
        ref[i // cpr, pl.ds((i % cpr) * 16, 16)] = z
        return 0

    lax.fori_loop(0, rows * cpr, body, 0, unroll=4)


@functools.partial(
    pl.kernel,
    out_type=jax.ShapeDtypeStruct((NC, NPAD, DH), jnp.float32),
    mesh=_mesh,
    scratch_types=[
        pltpu.VMEM((G,), jnp.int32),
        pltpu.VMEM((G,), jnp.int32),
        pltpu.VMEM((G, DH), jnp.float32),
        pltpu.VMEM((TAIL,), jnp.int32),
        pltpu.VMEM((TAIL,), jnp.int32),
        pltpu.VMEM((TAIL, DH), jnp.float32),
        pltpu.VMEM((ZR, DH), jnp.float32),
        pltpu.VMEM_SHARED((NPAD, DH), jnp.float32),
        pltpu.SemaphoreType.DMA,
    ],
)
def _segsum(h2, src, dst, out, src_v, dst_v, rows_v, tsrc_v, tdst_v,
            trows_v, zbuf, accum, sem):
    c = lax.axis_index("c")
    s = lax.axis_index("s")

    _zero_fill2(zbuf, ZR, DH)
    for k in range(ROWS_T // ZR):
        pltpu.sync_copy(zbuf, accum.at[pl.ds(s * ROWS_T + k * ZR, ZR)])
    plsc.subcore_barrier()

    ebase = s * EPT

    def chunk(k, _):
        base = pl.multiple_of(ebase + k * G, 8)
        pltpu.sync_copy(src.at[pl.ds(base, G)], src_v)
        pltpu.async_copy(h2.at[c].at[src_v], rows_v, sem).wait()
        pltpu.sync_copy(dst.at[pl.ds(base, G)], dst_v)
        pltpu.sync_copy(rows_v, accum.at[dst_v], add=True)
        return 0

    lax.fori_loop(0, NFULL, chunk, 0)

    tbase = pl.multiple_of(ebase + NFULL * G, 8)
    pltpu.sync_copy(src.at[pl.ds(tbase, TAIL)], tsrc_v)
    pltpu.async_copy(h2.at[c].at[tsrc_v], trows_v, sem).wait()
    pltpu.sync_copy(dst.at[pl.ds(tbase, TAIL)], tdst_v)
    pltpu.sync_copy(trows_v, accum.at[tdst_v], add=True)

    plsc.subcore_barrier()
    pltpu.sync_copy(accum.at[pl.ds(s * ROWS_T, ROWS_T)],
                    out.at[c, pl.ds(s * ROWS_T, ROWS_T)])


CW = 128


@functools.partial(
    pl.kernel,
    out_type=jax.ShapeDtypeStruct((NC, NPAD, CW), jnp.float32),
    mesh=_mesh,
    scratch_types=[
        pltpu.VMEM((G,), jnp.int32),
        pltpu.VMEM((G, CW), jnp.float32),
        pltpu.VMEM((TAIL_C,), jnp.int32),
        pltpu.VMEM((TAIL_C, CW), jnp.float32),
        pltpu.VMEM((ZR, CW), jnp.float32),
        pltpu.VMEM_SHARED((NPAD, CW), jnp.float32),
    ],
)
def _counts(dst, out, dst_v, ones_v, tdst_v, tones_v, zbuf, accum):
    c = lax.axis_index("c")
    s = lax.axis_index("s")
    rpt = NPAD // NS

    _zero_fill2(zbuf, ZR, CW)
    for k in range(rpt // ZR):
        pltpu.sync_copy(zbuf, accum.at[pl.ds(s * rpt + k * ZR, ZR)])

    one = jnp.ones((16,), jnp.float32)
    cpr = CW // 16

    def fill_ones(ref, n):
        def body(i, _):
            ref[i // cpr, pl.ds((i % cpr) * 16, 16)] = one
            return 0
        lax.fori_loop(0, n * cpr, body, 0, unroll=4)

    fill_ones(ones_v, G)
    fill_ones(tones_v, TAIL_C)
    plsc.subcore_barrier()

    ebase = c * (E // NC) + s * EPT_C

    def chunk(k, _):
        base = pl.multiple_of(ebase + k * G, 8)
        pltpu.sync_copy(dst.at[pl.ds(base, G)], dst_v)
        pltpu.sync_copy(ones_v, accum.at[dst_v], add=True)
        return 0

    lax.fori_loop(0, NFULL_C, chunk, 0)

    tbase = pl.multiple_of(ebase + NFULL_C * G, 8)
    pltpu.sync_copy(dst.at[pl.ds(tbase, TAIL_C)], tdst_v)
    pltpu.sync_copy(tones_v, accum.at[tdst_v], add=True)

    plsc.subcore_barrier()
    pltpu.sync_copy(accum.at[pl.ds(s * rpt, rpt)],
                    out.at[c, pl.ds(s * rpt, rpt)])


R = 400


def _tc_body(last, sum_ref, cnt_ref, h_ref, wl_ref, wr_ref, b_ref, out_ref):
    cnt = cnt_ref[0, :, 0] + cnt_ref[1, :, 0]
    inv = 1.0 / jnp.maximum(cnt, 1.0)
    m0 = sum_ref[0] * inv[:, None]
    m1 = sum_ref[1] * inv[:, None]
    acc = (
        jnp.dot(m0, wl_ref[0:DH, :], preferred_element_type=jnp.float32)
        + jnp.dot(m1, wl_ref[DH:D, :], preferred_element_type=jnp.float32)
        + jnp.dot(h_ref[0], wr_ref[0:DH, :], preferred_element_type=jnp.float32)
        + jnp.dot(h_ref[1], wr_ref[DH:D, :], preferred_element_type=jnp.float32)
        + b_ref[:][None, :]
    )
    if last:
        out_ref[:, :] = acc
    else:
        acc = jnp.maximum(acc, 0.0)
        out_ref[0] = acc[:, 0:DH]
        out_ref[1] = acc[:, DH:D]


def _tc_layer(summed2, cnt3, h2, wl, wr, b, last):
    in_specs = [
        pl.BlockSpec((NC, R, DH), lambda i: (0, i, 0)),
        pl.BlockSpec((NC, R, 1), lambda i: (0, i, 0)),
        pl.BlockSpec((NC, R, DH), lambda i: (0, i, 0)),
        pl.BlockSpec((D, D), lambda i: (0, 0)),
        pl.BlockSpec((D, D), lambda i: (0, 0)),
        pl.BlockSpec((D,), lambda i: (0,)),
    ]
    if last:
        out_shape = jax.ShapeDtypeStruct((N, D), jnp.float32)
        out_spec = pl.BlockSpec((R, D), lambda i: (i, 0))
    else:
        out_shape = jax.ShapeDtypeStruct((NC, N, DH), jnp.float32)
        out_spec = pl.BlockSpec((NC, R, DH), lambda i: (0, i, 0))
    return pl.pallas_call(
        functools.partial(_tc_body, last),
        grid=(N // R,),
        in_specs=in_specs,
        out_specs=out_spec,
        out_shape=out_shape,
    )(summed2, cnt3, h2, wl, wr, b)


def kernel(x, edge_index, Wl0, Wr0, b0, Wl1, Wr1, b1, Wl2, Wr2, b2,
           Wl3, Wr3, b3):
    src = edge_index[0].astype(jnp.int32)
    dst = edge_index[1].astype(jnp.int32)

    cnt3 = _counts(dst)[:, :N, 0:1]
    h2 = x.reshape(N, NC, DH).transpose(1, 0, 2)

    for wl, wr, b, last in ((Wl0, Wr0, b0, False),
                            (Wl1, Wr1, b1, False),
                            (Wl2, Wr2, b2, False),
                            (Wl3, Wr3, b3, True)):
        summed2 = _segsum(h2, src, dst)
        h2 = _tc_layer(summed2, cnt3, h2, wl, wr, b, last)
    return h2
